# baseline (speedup 1.0000x reference)
import functools

import jax
import jax.numpy as jnp
from jax import lax
from jax.experimental import pallas as pl
from jax.experimental.pallas import tpu as pltpu
from jax.experimental.pallas import tpu_sc as plsc

B = 2048
L = 160
K = 16
NCORES = 2
NSUB = 16
NWORKERS = NCORES * NSUB
ROWS_PER_W = B // NWORKERS
CHUNKS_PER_ROW = L // 16
BBLK = 512
GRID = B // BBLK

_SQRT_HALF = 0.7071067811865476

def _gelu(x):
    return 0.5 * x * (1.0 + lax.erf(x * _SQRT_HALF))


def _softplus(x):
    return jnp.maximum(x, 0.0) + jnp.log(1.0 + jnp.exp(-jnp.abs(x)))


def _silu(x):
    return x * (1.0 / (1.0 + jnp.exp(-x)))


def _tc_body(x_ref, c0w, c0b, c1w, c1b, c2w, c2b, c3w, c3b, c4w, c4b,
             pos_ref, lnb_ref, ipw_ref, mcw_ref, mcb_ref, xpw_ref,
             dtw_ref, dtb_ref, acat_ref, d_ref, opw_ref, emb_ref,
             res2_ref, m_ref, embb_ref, *wband):
    cws = (c0w, c1w, c2w, c3w, c4w)
    cbs = (c0b, c1b, c2b, c3b, c4b)
    f32 = jnp.float32
    dils = (1, 2, 4, 8, 16)

    @pl.when(pl.program_id(0) == 0)
    def _build_bands():
        ri = lax.broadcasted_iota(jnp.int32, (L, L), 0)
        ci = lax.broadcasted_iota(jnp.int32, (L, L), 1)
        diff = ri - ci
        for li in range(5):
            d = dils[li]
            w = jnp.zeros((L, L), f32)
            for k in range(5):
                w = jnp.where(diff == (k - 2) * d, cws[li][0, 0, k], w)
            wband[li][...] = w

    x = x_ref[...]
    h = x
    for li in range(5):
        acc = jnp.dot(h, wband[li][...], preferred_element_type=f32)
        acc = acc + cbs[li][0]
        h = _gelu(acc) if li < 4 else acc
    res2_ref[...] = h + x

    @pl.when(pl.program_id(0) == 0)
    def _compute_m():
        e_col = emb_ref[...]
        i16r = lax.broadcasted_iota(jnp.int32, (K, K), 1)
        i16c = lax.broadcasted_iota(jnp.int32, (K, K), 0)
        ec = jnp.broadcast_to(e_col, (K, K))
        ident = jnp.where(i16r == i16c, 1.0, 0.0)
        e_row = lax.dot_general(e_col, ident, (((0,), (0,)), ((), ())),
                                preferred_element_type=f32)
        er = jnp.broadcast_to(e_row, (K, K))
        smaller = (er < ec) | ((er == ec) & (i16r < i16c))
        rank = jnp.sum(jnp.where(smaller, 1, 0), axis=1,
                       keepdims=True)
        msel = jnp.where(rank == lax.broadcasted_iota(jnp.int32, (K, K), 1),
                         1.0, 0.0)
        se_col = lax.dot_general(msel, e_col, (((0,), (0,)), ((), ())),
                                 preferred_element_type=f32)
        ii_col = lax.broadcasted_iota(jnp.int32, (K, 1), 0).astype(f32)
        si_col = lax.dot_general(msel, ii_col, (((0,), (0,)), ((), ())),
                                 preferred_element_type=f32)
        nxt_se = jnp.concatenate([se_col[1:, :], se_col[15:16, :]], axis=0)
        nxt_si = jnp.concatenate([si_col[1:, :], si_col[15:16, :]], axis=0)
        mid_col = 0.5 * (se_col + nxt_se)
        dq_col = nxt_se - se_col
        di_col = nxt_si - si_col
        embb_ref[...] = jnp.concatenate([
            jnp.broadcast_to(mid_col, (K, 16)),
            jnp.broadcast_to(dq_col, (K, 16)),
            jnp.broadcast_to(di_col, (K, 16)),
            jnp.broadcast_to(se_col[0:1, :], (1, 16)),
            jnp.broadcast_to(si_col[0:1, :], (1, 16)),
        ], axis=0)
        lnb = lnb_ref[0]
        xc0 = lnb * ipw_ref[0, 0]
        xc1 = lnb * ipw_ref[1, 0]
        zc0 = lnb * ipw_ref[2, 0]
        zc1 = lnb * ipw_ref[3, 0]
        tt = lax.broadcasted_iota(jnp.int32, (1, L), 1)

        def urow(dch, xc):
            w1 = mcw_ref[dch, 0, 1]
            w2 = mcw_ref[dch, 0, 2]
            w3 = mcw_ref[dch, 0, 3]
            sfull = mcw_ref[dch, 0, 0] + w1 + w2 + w3
            s = jnp.where(tt == 0, w3,
                          jnp.where(tt == 1, w2 + w3,
                                    jnp.where(tt == 2, w1 + w2 + w3, sfull)))
            xcr = s * xc + mcb_ref[dch]
            return _silu(xcr)

        u0 = urow(0, xc0)
        u1 = urow(1, xc1)
        xdbl = xpw_ref[:, 0:1] * u0 + xpw_ref[:, 1:2] * u1
        dtr = xdbl[0:1, :]
        Bm = xdbl[1:49, :]
        Cm = xdbl[49:97, :]
        dlt0 = _softplus(dtr * dtw_ref[0, 0] + dtb_ref[0])
        dlt1 = _softplus(dtr * dtw_ref[1, 0] + dtb_ref[1])
        d48_0 = jnp.broadcast_to(dlt0, (48, L))
        d48_1 = jnp.broadcast_to(dlt1, (48, L))
        dcat = jnp.concatenate([d48_0, d48_1, d48_0, d48_1], axis=0)
        u48_0 = jnp.broadcast_to(u0, (48, L))
        u48_1 = jnp.broadcast_to(u1, (48, L))
        ucat = jnp.concatenate([u48_0, u48_1, u48_0, u48_1], axis=0)
        a_coef = -jnp.exp(acat_ref[...])
        a = jnp.exp(dcat * a_coef)
        bt = jnp.concatenate([Bm, Bm, Bm, Bm], axis=0)
        b = dcat * bt * ucat
        s_ = 1
        while s_ < L:
            pad1 = jnp.ones((192, s_), f32)
            pad0 = jnp.zeros((192, s_), f32)
            a_sh = jnp.concatenate([pad1, a[:, :L - s_]], axis=1)
            b_sh = jnp.concatenate([pad0, b[:, :L - s_]], axis=1)
            b = a * b_sh + b
            a = a * a_sh
            s_ *= 2
        y0 = jnp.sum(b[0:48, :] * Cm, axis=0, keepdims=True)
        y1 = jnp.sum(b[48:96, :] * Cm, axis=0, keepdims=True)
        y2 = jnp.sum(b[96:144, :] * Cm, axis=0, keepdims=True)
        y3 = jnp.sum(b[144:192, :] * Cm, axis=0, keepdims=True)
        sz0 = _silu(jnp.full((1, 1), zc0, f32))
        sz1 = _silu(jnp.full((1, 1), zc1, f32))
        w0 = opw_ref[0, 0]
        w1 = opw_ref[0, 1]
        yf0 = (y0 + u0 * d_ref[0]) * sz0
        yf1 = (y1 + u1 * d_ref[1]) * sz1
        yb0 = (y2 + u0 * d_ref[0]) * sz0
        yb1 = (y3 + u1 * d_ref[1]) * sz1
        ri = lax.broadcasted_iota(jnp.int32, (L, L), 0)
        ci = lax.broadcasted_iota(jnp.int32, (L, L), 1)
        perm = jnp.where(ri + ci == L - 1, 1.0, 0.0)
        yb2 = jnp.concatenate([yb0, yb1], axis=0)
        ybr = jnp.dot(yb2, perm, preferred_element_type=f32)
        m2 = (yf0 + ybr[0:1, :]) * w0 + (yf1 + ybr[1:2, :]) * w1
        m_ref[...] = jnp.concatenate([m2, pos_ref[...]], axis=0)


def _sc_body(x_hbm, m_hbm, emb_hbm, quant_hbm, idx_hbm, loss_hbm,
             xb, mb, eb, qb, ib, lb):
    cid = lax.axis_index("c")
    sid = lax.axis_index("s")
    wid = sid * NCORES + cid
    row0 = wid * ROWS_PER_W
    pltpu.sync_copy(x_hbm.at[pl.ds(row0, ROWS_PER_W)], xb)
    pltpu.sync_copy(m_hbm, mb)
    pltpu.sync_copy(emb_hbm, eb)
    mrow = [mb[0, pl.ds(c * 16, 16)] for c in range(CHUNKS_PER_ROW)]
    prow = [mb[1, pl.ds(c * 16, 16)] for c in range(CHUNKS_PER_ROW)]
    mids = [eb[j] for j in range(K - 1)]
    dqs = [eb[K + j] for j in range(K - 1)]
    dis = [eb[2 * K + j] for j in range(K - 1)]
    q0 = eb[3 * K]
    i0 = eb[3 * K + 1]
    zf = jnp.zeros((16,), jnp.float32)

    def body(r, acc):
        for c in range(CHUNKS_PER_ROW):
            ds = pl.ds(c * 16, 16)
            xv = mrow[c] + (xb[r, ds] + prow[c])
            qv = q0
            ivf = i0
            for j in range(K - 1):
                mask = xv > mids[j]
                qv = qv + jnp.where(mask, dqs[j], zf)
                ivf = ivf + jnp.where(mask, dis[j], zf)
            qb[r, ds] = qv
            ib[r, ds] = ivf.astype(jnp.int32)
            df = qv - xv
            acc = acc + df * df
        return acc

    acc = lax.fori_loop(0, ROWS_PER_W, body, jnp.zeros((16,), jnp.float32))
    lb[...] = acc
    pltpu.sync_copy(qb, quant_hbm.at[pl.ds(row0, ROWS_PER_W)])
    pltpu.sync_copy(ib, idx_hbm.at[pl.ds(row0, ROWS_PER_W)])
    pltpu.sync_copy(lb, loss_hbm.at[wid])


def _smem_spec():
    return pl.BlockSpec(memory_space=pltpu.SMEM)


def _full_vmem(shape):
    return pl.BlockSpec(shape, lambda *_: tuple(0 for _ in shape))


_tc_call = pl.pallas_call(
    _tc_body,
    grid=(GRID,),
    in_specs=[
        pl.BlockSpec((BBLK, L), lambda i: (i, 0)),
        _smem_spec(), _smem_spec(),
        _smem_spec(), _smem_spec(),
        _smem_spec(), _smem_spec(),
        _smem_spec(), _smem_spec(),
        _smem_spec(), _smem_spec(),
        _full_vmem((1, L)),
        _smem_spec(),
        _smem_spec(),
        _smem_spec(),
        _smem_spec(),
        _full_vmem((97, 2)),
        _smem_spec(),
        _smem_spec(),
        _full_vmem((192, 1)),
        _smem_spec(),
        _smem_spec(),
        _full_vmem((K, 1)),
    ],
    out_specs=[
        pl.BlockSpec((BBLK, L), lambda i: (i, 0)),
        pl.BlockSpec((2, L), lambda i: (0, 0)),
        pl.BlockSpec((3 * K + 2, 16), lambda i: (0, 0)),
    ],
    out_shape=[
        jax.ShapeDtypeStruct((B, L), jnp.float32),
        jax.ShapeDtypeStruct((2, L), jnp.float32),
        jax.ShapeDtypeStruct((3 * K + 2, 16), jnp.float32),
    ],
    scratch_shapes=[pltpu.VMEM((L, L), jnp.float32) for _ in range(5)],
)


@functools.cache
def _get_sc_call():
    mesh = plsc.VectorSubcoreMesh(core_axis_name="c", subcore_axis_name="s",
                                  num_cores=NCORES, num_subcores=NSUB)
    return pl.kernel(
        _sc_body,
        out_type=[
            jax.ShapeDtypeStruct((B, L), jnp.float32),
            jax.ShapeDtypeStruct((B, L), jnp.int32),
            jax.ShapeDtypeStruct((NWORKERS, 16), jnp.float32),
        ],
        mesh=mesh,
        scratch_types=[
            pltpu.VMEM((ROWS_PER_W, L), jnp.float32),
            pltpu.VMEM((2, L), jnp.float32),
            pltpu.VMEM((3 * K + 2, 16), jnp.float32),
            pltpu.VMEM((ROWS_PER_W, L), jnp.float32),
            pltpu.VMEM((ROWS_PER_W, L), jnp.int32),
            pltpu.VMEM((16,), jnp.float32),
        ],
    )


def kernel(inputs, c0_w, c0_b, c1_w, c1_b, c2_w, c2_b, c3_w, c3_b, c4_w,
           c4_b, pos_emb, ln_w, ln_b, in_proj_w, conv1d_w, conv1d_b,
           x_proj_w, dt_proj_w, dt_proj_b, A_log, A_b_log, D, out_proj_w,
           emb):
    del ln_w
    acat = jnp.concatenate(
        [A_log.reshape(-1), A_b_log.reshape(-1)]).reshape(2 * 2 * 48, 1)
    res2, m_plus, embb = _tc_call(
        inputs,
        c0_w, c0_b, c1_w, c1_b, c2_w, c2_b, c3_w, c3_b, c4_w, c4_b,
        pos_emb.reshape(1, L), ln_b, in_proj_w, conv1d_w, conv1d_b,
        x_proj_w, dt_proj_w, dt_proj_b, acat, D, out_proj_w, emb)
    quant, idx, loss_part = _get_sc_call()(res2, m_plus, embb)
    c_loss = 0.5 * (jnp.sum(loss_part) / (B * L))
    return c_loss[None], quant, idx


# speedup vs baseline: 1.4561x; 1.4561x over previous
import functools

import jax
import jax.numpy as jnp
from jax import lax
from jax.experimental import pallas as pl
from jax.experimental.pallas import tpu as pltpu
from jax.experimental.pallas import tpu_sc as plsc

B = 2048
L = 160
K = 16
NCORES = 2
NSUB = 16
NWORKERS = NCORES * NSUB
ROWS_PER_W = B // NWORKERS
CHUNKS_PER_ROW = L // 16
BBLK = 512
GRID = B // BBLK

_SQRT_HALF = 0.7071067811865476

def _gelu(x):
    return 0.5 * x * (1.0 + lax.erf(x * _SQRT_HALF))


def _softplus(x):
    return jnp.maximum(x, 0.0) + jnp.log(1.0 + jnp.exp(-jnp.abs(x)))


def _silu(x):
    return x * (1.0 / (1.0 + jnp.exp(-x)))


def _tc_body(x_ref, c0w, c0b, c1w, c1b, c2w, c2b, c3w, c3b, c4w, c4b,
             pos_ref, lnb_ref, ipw_ref, mcw_ref, mcb_ref, xpw_ref,
             dtw_ref, dtb_ref, acat_ref, d_ref, opw_ref, emb_ref,
             res2_ref, m_ref, embb_ref, *wband):
    cws = (c0w, c1w, c2w, c3w, c4w)
    cbs = (c0b, c1b, c2b, c3b, c4b)
    f32 = jnp.float32
    dils = (1, 2, 4, 8, 16)

    @pl.when(pl.program_id(0) == 0)
    def _build_bands():
        ri = lax.broadcasted_iota(jnp.int32, (L, L), 0)
        ci = lax.broadcasted_iota(jnp.int32, (L, L), 1)
        diff = ri - ci
        for li in range(5):
            d = dils[li]
            w = jnp.zeros((L, L), f32)
            for k in range(5):
                w = jnp.where(diff == (k - 2) * d, cws[li][0, 0, k], w)
            wband[li][...] = w

    x = x_ref[...]
    h = x
    for li in range(5):
        acc = jnp.dot(h, wband[li][...], preferred_element_type=f32)
        acc = acc + cbs[li][0]
        h = _gelu(acc) if li < 4 else acc
    res2_ref[...] = h + x

    @pl.when(pl.program_id(0) == 0)
    def _compute_m():
        embb_ref[...] = jnp.broadcast_to(emb_ref[...], (K, 16))
        lnb = lnb_ref[0]
        xc0 = lnb * ipw_ref[0, 0]
        xc1 = lnb * ipw_ref[1, 0]
        zc0 = lnb * ipw_ref[2, 0]
        zc1 = lnb * ipw_ref[3, 0]
        tt = lax.broadcasted_iota(jnp.int32, (1, L), 1)

        def urow(dch, xc):
            w1 = mcw_ref[dch, 0, 1]
            w2 = mcw_ref[dch, 0, 2]
            w3 = mcw_ref[dch, 0, 3]
            sfull = mcw_ref[dch, 0, 0] + w1 + w2 + w3
            s = jnp.where(tt == 0, w3,
                          jnp.where(tt == 1, w2 + w3,
                                    jnp.where(tt == 2, w1 + w2 + w3, sfull)))
            xcr = s * xc + mcb_ref[dch]
            return _silu(xcr)

        u0 = urow(0, xc0)
        u1 = urow(1, xc1)
        xdbl = xpw_ref[:, 0:1] * u0 + xpw_ref[:, 1:2] * u1
        dtr = xdbl[0:1, :]
        Bm = xdbl[1:49, :]
        Cm = xdbl[49:97, :]
        dlt0 = _softplus(dtr * dtw_ref[0, 0] + dtb_ref[0])
        dlt1 = _softplus(dtr * dtw_ref[1, 0] + dtb_ref[1])
        d48_0 = jnp.broadcast_to(dlt0, (48, L))
        d48_1 = jnp.broadcast_to(dlt1, (48, L))
        dcat = jnp.concatenate([d48_0, d48_1, d48_0, d48_1], axis=0)
        u48_0 = jnp.broadcast_to(u0, (48, L))
        u48_1 = jnp.broadcast_to(u1, (48, L))
        ucat = jnp.concatenate([u48_0, u48_1, u48_0, u48_1], axis=0)
        a_coef = -jnp.exp(acat_ref[...])
        a = jnp.exp(dcat * a_coef)
        bt = jnp.concatenate([Bm, Bm, Bm, Bm], axis=0)
        b = dcat * bt * ucat
        s_ = 1
        while s_ < L:
            pad1 = jnp.ones((192, s_), f32)
            pad0 = jnp.zeros((192, s_), f32)
            a_sh = jnp.concatenate([pad1, a[:, :L - s_]], axis=1)
            b_sh = jnp.concatenate([pad0, b[:, :L - s_]], axis=1)
            b = a * b_sh + b
            a = a * a_sh
            s_ *= 2
        y0 = jnp.sum(b[0:48, :] * Cm, axis=0, keepdims=True)
        y1 = jnp.sum(b[48:96, :] * Cm, axis=0, keepdims=True)
        y2 = jnp.sum(b[96:144, :] * Cm, axis=0, keepdims=True)
        y3 = jnp.sum(b[144:192, :] * Cm, axis=0, keepdims=True)
        sz0 = _silu(jnp.full((1, 1), zc0, f32))
        sz1 = _silu(jnp.full((1, 1), zc1, f32))
        w0 = opw_ref[0, 0]
        w1 = opw_ref[0, 1]
        yf0 = (y0 + u0 * d_ref[0]) * sz0
        yf1 = (y1 + u1 * d_ref[1]) * sz1
        yb0 = (y2 + u0 * d_ref[0]) * sz0
        yb1 = (y3 + u1 * d_ref[1]) * sz1
        ri = lax.broadcasted_iota(jnp.int32, (L, L), 0)
        ci = lax.broadcasted_iota(jnp.int32, (L, L), 1)
        perm = jnp.where(ri + ci == L - 1, 1.0, 0.0)
        yb2 = jnp.concatenate([yb0, yb1], axis=0)
        ybr = jnp.dot(yb2, perm, preferred_element_type=f32)
        m2 = (yf0 + ybr[0:1, :]) * w0 + (yf1 + ybr[1:2, :]) * w1
        m_ref[...] = jnp.concatenate([m2, pos_ref[...]], axis=0)


def _sc_body(x_hbm, m_hbm, emb_hbm, quant_hbm, idx_hbm, loss_hbm,
             xb, mb, eb, qb, ib, lb):
    cid = lax.axis_index("c")
    sid = lax.axis_index("s")
    wid = sid * NCORES + cid
    row0 = wid * ROWS_PER_W
    pltpu.sync_copy(x_hbm.at[pl.ds(row0, ROWS_PER_W)], xb)
    pltpu.sync_copy(m_hbm, mb)
    pltpu.sync_copy(emb_hbm, eb)
    mrow = [mb[0, pl.ds(c * 16, 16)] for c in range(CHUNKS_PER_ROW)]
    prow = [mb[1, pl.ds(c * 16, 16)] for c in range(CHUNKS_PER_ROW)]

    def body(r, acc):
        for c in range(CHUNKS_PER_ROW):
            ds = pl.ds(c * 16, 16)
            xv = mrow[c] + (xb[r, ds] + prow[c])
            e0 = eb[0]
            bd = jnp.abs(xv - e0)
            bi = jnp.zeros((16,), jnp.int32)
            bq = e0
            for k in range(1, K):
                ek = eb[k]
                dk = jnp.abs(xv - ek)
                bet = dk < bd
                bd = jnp.where(bet, dk, bd)
                bi = jnp.where(bet, jnp.full((16,), k, jnp.int32), bi)
                bq = jnp.where(bet, ek, bq)
            qb[r, ds] = bq
            ib[r, ds] = bi
            df = bq - xv
            acc = acc + df * df
        return acc

    acc = lax.fori_loop(0, ROWS_PER_W, body, jnp.zeros((16,), jnp.float32))
    lb[...] = acc
    pltpu.sync_copy(qb, quant_hbm.at[pl.ds(row0, ROWS_PER_W)])
    pltpu.sync_copy(ib, idx_hbm.at[pl.ds(row0, ROWS_PER_W)])
    pltpu.sync_copy(lb, loss_hbm.at[wid])


def _smem_spec():
    return pl.BlockSpec(memory_space=pltpu.SMEM)


def _full_vmem(shape):
    return pl.BlockSpec(shape, lambda *_: tuple(0 for _ in shape))


_tc_call = pl.pallas_call(
    _tc_body,
    grid=(GRID,),
    in_specs=[
        pl.BlockSpec((BBLK, L), lambda i: (i, 0)),
        _smem_spec(), _smem_spec(),
        _smem_spec(), _smem_spec(),
        _smem_spec(), _smem_spec(),
        _smem_spec(), _smem_spec(),
        _smem_spec(), _smem_spec(),
        _full_vmem((1, L)),
        _smem_spec(),
        _smem_spec(),
        _smem_spec(),
        _smem_spec(),
        _full_vmem((97, 2)),
        _smem_spec(),
        _smem_spec(),
        _full_vmem((192, 1)),
        _smem_spec(),
        _smem_spec(),
        _full_vmem((K, 1)),
    ],
    out_specs=[
        pl.BlockSpec((BBLK, L), lambda i: (i, 0)),
        pl.BlockSpec((2, L), lambda i: (0, 0)),
        pl.BlockSpec((K, 16), lambda i: (0, 0)),
    ],
    out_shape=[
        jax.ShapeDtypeStruct((B, L), jnp.float32),
        jax.ShapeDtypeStruct((2, L), jnp.float32),
        jax.ShapeDtypeStruct((K, 16), jnp.float32),
    ],
    scratch_shapes=[pltpu.VMEM((L, L), jnp.float32) for _ in range(5)],
)


@functools.cache
def _get_sc_call():
    mesh = plsc.VectorSubcoreMesh(core_axis_name="c", subcore_axis_name="s",
                                  num_cores=NCORES, num_subcores=NSUB)
    return pl.kernel(
        _sc_body,
        out_type=[
            jax.ShapeDtypeStruct((B, L), jnp.float32),
            jax.ShapeDtypeStruct((B, L), jnp.int32),
            jax.ShapeDtypeStruct((NWORKERS, 16), jnp.float32),
        ],
        mesh=mesh,
        scratch_types=[
            pltpu.VMEM((ROWS_PER_W, L), jnp.float32),
            pltpu.VMEM((2, L), jnp.float32),
            pltpu.VMEM((K, 16), jnp.float32),
            pltpu.VMEM((ROWS_PER_W, L), jnp.float32),
            pltpu.VMEM((ROWS_PER_W, L), jnp.int32),
            pltpu.VMEM((16,), jnp.float32),
        ],
    )


def kernel(inputs, c0_w, c0_b, c1_w, c1_b, c2_w, c2_b, c3_w, c3_b, c4_w,
           c4_b, pos_emb, ln_w, ln_b, in_proj_w, conv1d_w, conv1d_b,
           x_proj_w, dt_proj_w, dt_proj_b, A_log, A_b_log, D, out_proj_w,
           emb):
    del ln_w
    acat = jnp.concatenate(
        [A_log.reshape(-1), A_b_log.reshape(-1)]).reshape(2 * 2 * 48, 1)
    res2, m_plus, embb = _tc_call(
        inputs,
        c0_w, c0_b, c1_w, c1_b, c2_w, c2_b, c3_w, c3_b, c4_w, c4_b,
        pos_emb.reshape(1, L), ln_b, in_proj_w, conv1d_w, conv1d_b,
        x_proj_w, dt_proj_w, dt_proj_b, acat, D, out_proj_w, emb)
    quant, idx, loss_part = _get_sc_call()(res2, m_plus, embb)
    c_loss = 0.5 * (jnp.sum(loss_part) / (B * L))
    return c_loss[None], quant, idx


# speedup vs baseline: 1.4735x; 1.0119x over previous
import functools

import jax
import jax.numpy as jnp
from jax import lax
from jax.experimental import pallas as pl
from jax.experimental.pallas import tpu as pltpu
from jax.experimental.pallas import tpu_sc as plsc

B = 2048
L = 160
K = 16
NCORES = 2
NSUB = 16
NWORKERS = NCORES * NSUB
ROWS_PER_W = B // NWORKERS
CHUNKS_PER_ROW = L // 16
BBLK = 1024
GRID = B // BBLK

_SQRT_HALF = 0.7071067811865476

def _gelu(x):
    return 0.5 * x * (1.0 + lax.erf(x * _SQRT_HALF))


def _softplus(x):
    return jnp.maximum(x, 0.0) + jnp.log(1.0 + jnp.exp(-jnp.abs(x)))


def _silu(x):
    return x * (1.0 / (1.0 + jnp.exp(-x)))


def _tc_body(x_ref, c0w, c0b, c1w, c1b, c2w, c2b, c3w, c3b, c4w, c4b,
             pos_ref, lnb_ref, ipw_ref, mcw_ref, mcb_ref, xpw_ref,
             dtw_ref, dtb_ref, acat_ref, d_ref, opw_ref, emb_ref,
             res2_ref, m_ref, embb_ref, *wband):
    cws = (c0w, c1w, c2w, c3w, c4w)
    cbs = (c0b, c1b, c2b, c3b, c4b)
    f32 = jnp.float32
    dils = (1, 2, 4, 8, 16)

    @pl.when(pl.program_id(0) == 0)
    def _build_bands():
        ri = lax.broadcasted_iota(jnp.int32, (L, L), 0)
        ci = lax.broadcasted_iota(jnp.int32, (L, L), 1)
        diff = ri - ci
        for li in range(5):
            d = dils[li]
            w = jnp.zeros((L, L), f32)
            for k in range(5):
                w = jnp.where(diff == (k - 2) * d, cws[li][0, 0, k], w)
            wband[li][...] = w

    x = x_ref[...]
    h = x
    for li in range(5):
        acc = jnp.dot(h, wband[li][...], preferred_element_type=f32)
        acc = acc + cbs[li][0]
        h = _gelu(acc) if li < 4 else acc
    res2_ref[...] = h + x

    @pl.when(pl.program_id(0) == 0)
    def _compute_m():
        embb_ref[...] = jnp.broadcast_to(emb_ref[...], (K, 16))
        lnb = lnb_ref[0]
        xc0 = lnb * ipw_ref[0, 0]
        xc1 = lnb * ipw_ref[1, 0]
        zc0 = lnb * ipw_ref[2, 0]
        zc1 = lnb * ipw_ref[3, 0]
        tt = lax.broadcasted_iota(jnp.int32, (1, L), 1)

        def urow(dch, xc):
            w1 = mcw_ref[dch, 0, 1]
            w2 = mcw_ref[dch, 0, 2]
            w3 = mcw_ref[dch, 0, 3]
            sfull = mcw_ref[dch, 0, 0] + w1 + w2 + w3
            s = jnp.where(tt == 0, w3,
                          jnp.where(tt == 1, w2 + w3,
                                    jnp.where(tt == 2, w1 + w2 + w3, sfull)))
            xcr = s * xc + mcb_ref[dch]
            return _silu(xcr)

        u0 = urow(0, xc0)
        u1 = urow(1, xc1)
        xdbl = xpw_ref[:, 0:1] * u0 + xpw_ref[:, 1:2] * u1
        dtr = xdbl[0:1, :]
        Bm = xdbl[1:49, :]
        Cm = xdbl[49:97, :]
        dlt0 = _softplus(dtr * dtw_ref[0, 0] + dtb_ref[0])
        dlt1 = _softplus(dtr * dtw_ref[1, 0] + dtb_ref[1])
        d48_0 = jnp.broadcast_to(dlt0, (48, L))
        d48_1 = jnp.broadcast_to(dlt1, (48, L))
        dcat = jnp.concatenate([d48_0, d48_1, d48_0, d48_1], axis=0)
        u48_0 = jnp.broadcast_to(u0, (48, L))
        u48_1 = jnp.broadcast_to(u1, (48, L))
        ucat = jnp.concatenate([u48_0, u48_1, u48_0, u48_1], axis=0)
        a_coef = -jnp.exp(acat_ref[...])
        a = jnp.exp(dcat * a_coef)
        bt = jnp.concatenate([Bm, Bm, Bm, Bm], axis=0)
        b = dcat * bt * ucat
        s_ = 1
        while s_ < L:
            pad1 = jnp.ones((192, s_), f32)
            pad0 = jnp.zeros((192, s_), f32)
            a_sh = jnp.concatenate([pad1, a[:, :L - s_]], axis=1)
            b_sh = jnp.concatenate([pad0, b[:, :L - s_]], axis=1)
            b = a * b_sh + b
            a = a * a_sh
            s_ *= 2
        y0 = jnp.sum(b[0:48, :] * Cm, axis=0, keepdims=True)
        y1 = jnp.sum(b[48:96, :] * Cm, axis=0, keepdims=True)
        y2 = jnp.sum(b[96:144, :] * Cm, axis=0, keepdims=True)
        y3 = jnp.sum(b[144:192, :] * Cm, axis=0, keepdims=True)
        sz0 = _silu(jnp.full((1, 1), zc0, f32))
        sz1 = _silu(jnp.full((1, 1), zc1, f32))
        w0 = opw_ref[0, 0]
        w1 = opw_ref[0, 1]
        yf0 = (y0 + u0 * d_ref[0]) * sz0
        yf1 = (y1 + u1 * d_ref[1]) * sz1
        yb0 = (y2 + u0 * d_ref[0]) * sz0
        yb1 = (y3 + u1 * d_ref[1]) * sz1
        ri = lax.broadcasted_iota(jnp.int32, (L, L), 0)
        ci = lax.broadcasted_iota(jnp.int32, (L, L), 1)
        perm = jnp.where(ri + ci == L - 1, 1.0, 0.0)
        yb2 = jnp.concatenate([yb0, yb1], axis=0)
        ybr = jnp.dot(yb2, perm, preferred_element_type=f32)
        m2 = (yf0 + ybr[0:1, :]) * w0 + (yf1 + ybr[1:2, :]) * w1
        m_ref[...] = jnp.concatenate([m2, pos_ref[...]], axis=0)


def _sc_body(x_hbm, m_hbm, emb_hbm, quant_hbm, idx_hbm, loss_hbm,
             xb, mb, eb, qb, ib, lb):
    cid = lax.axis_index("c")
    sid = lax.axis_index("s")
    wid = sid * NCORES + cid
    row0 = wid * ROWS_PER_W
    pltpu.sync_copy(x_hbm.at[pl.ds(row0, ROWS_PER_W)], xb)
    pltpu.sync_copy(m_hbm, mb)
    pltpu.sync_copy(emb_hbm, eb)
    mrow = [mb[0, pl.ds(c * 16, 16)] for c in range(CHUNKS_PER_ROW)]
    prow = [mb[1, pl.ds(c * 16, 16)] for c in range(CHUNKS_PER_ROW)]

    def body(r, acc):
        for c in range(CHUNKS_PER_ROW):
            ds = pl.ds(c * 16, 16)
            xv = mrow[c] + (xb[r, ds] + prow[c])
            e0 = eb[0]
            bd = jnp.abs(xv - e0)
            bi = jnp.zeros((16,), jnp.int32)
            bq = e0
            for k in range(1, K):
                ek = eb[k]
                dk = jnp.abs(xv - ek)
                bet = dk < bd
                bd = jnp.where(bet, dk, bd)
                bi = jnp.where(bet, jnp.full((16,), k, jnp.int32), bi)
                bq = jnp.where(bet, ek, bq)
            qb[r, ds] = bq
            ib[r, ds] = bi
            df = bq - xv
            acc = acc + df * df
        return acc

    acc = lax.fori_loop(0, ROWS_PER_W, body, jnp.zeros((16,), jnp.float32))
    lb[...] = acc
    pltpu.sync_copy(qb, quant_hbm.at[pl.ds(row0, ROWS_PER_W)])
    pltpu.sync_copy(ib, idx_hbm.at[pl.ds(row0, ROWS_PER_W)])
    pltpu.sync_copy(lb, loss_hbm.at[wid])


def _smem_spec():
    return pl.BlockSpec(memory_space=pltpu.SMEM)


def _full_vmem(shape):
    return pl.BlockSpec(shape, lambda *_: tuple(0 for _ in shape))


_tc_call = pl.pallas_call(
    _tc_body,
    grid=(GRID,),
    in_specs=[
        pl.BlockSpec((BBLK, L), lambda i: (i, 0)),
        _smem_spec(), _smem_spec(),
        _smem_spec(), _smem_spec(),
        _smem_spec(), _smem_spec(),
        _smem_spec(), _smem_spec(),
        _smem_spec(), _smem_spec(),
        _full_vmem((1, L)),
        _smem_spec(),
        _smem_spec(),
        _smem_spec(),
        _smem_spec(),
        _full_vmem((97, 2)),
        _smem_spec(),
        _smem_spec(),
        _full_vmem((192, 1)),
        _smem_spec(),
        _smem_spec(),
        _full_vmem((K, 1)),
    ],
    out_specs=[
        pl.BlockSpec((BBLK, L), lambda i: (i, 0)),
        pl.BlockSpec((2, L), lambda i: (0, 0)),
        pl.BlockSpec((K, 16), lambda i: (0, 0)),
    ],
    out_shape=[
        jax.ShapeDtypeStruct((B, L), jnp.float32),
        jax.ShapeDtypeStruct((2, L), jnp.float32),
        jax.ShapeDtypeStruct((K, 16), jnp.float32),
    ],
    scratch_shapes=[pltpu.VMEM((L, L), jnp.float32) for _ in range(5)],
)


@functools.cache
def _get_sc_call():
    mesh = plsc.VectorSubcoreMesh(core_axis_name="c", subcore_axis_name="s",
                                  num_cores=NCORES, num_subcores=NSUB)
    return pl.kernel(
        _sc_body,
        out_type=[
            jax.ShapeDtypeStruct((B, L), jnp.float32),
            jax.ShapeDtypeStruct((B, L), jnp.int32),
            jax.ShapeDtypeStruct((NWORKERS, 16), jnp.float32),
        ],
        mesh=mesh,
        scratch_types=[
            pltpu.VMEM((ROWS_PER_W, L), jnp.float32),
            pltpu.VMEM((2, L), jnp.float32),
            pltpu.VMEM((K, 16), jnp.float32),
            pltpu.VMEM((ROWS_PER_W, L), jnp.float32),
            pltpu.VMEM((ROWS_PER_W, L), jnp.int32),
            pltpu.VMEM((16,), jnp.float32),
        ],
    )


def kernel(inputs, c0_w, c0_b, c1_w, c1_b, c2_w, c2_b, c3_w, c3_b, c4_w,
           c4_b, pos_emb, ln_w, ln_b, in_proj_w, conv1d_w, conv1d_b,
           x_proj_w, dt_proj_w, dt_proj_b, A_log, A_b_log, D, out_proj_w,
           emb):
    del ln_w
    acat = jnp.concatenate(
        [A_log.reshape(-1), A_b_log.reshape(-1)]).reshape(2 * 2 * 48, 1)
    res2, m_plus, embb = _tc_call(
        inputs,
        c0_w, c0_b, c1_w, c1_b, c2_w, c2_b, c3_w, c3_b, c4_w, c4_b,
        pos_emb.reshape(1, L), ln_b, in_proj_w, conv1d_w, conv1d_b,
        x_proj_w, dt_proj_w, dt_proj_b, acat, D, out_proj_w, emb)
    quant, idx, loss_part = _get_sc_call()(res2, m_plus, embb)
    c_loss = 0.5 * (jnp.sum(loss_part) / (B * L))
    return c_loss[None], quant, idx


# speedup vs baseline: 1.4911x; 1.0119x over previous
import functools

import jax
import jax.numpy as jnp
from jax import lax
from jax.experimental import pallas as pl
from jax.experimental.pallas import tpu as pltpu
from jax.experimental.pallas import tpu_sc as plsc

B = 2048
L = 160
K = 16
NCORES = 2
NSUB = 16
NWORKERS = NCORES * NSUB
ROWS_PER_W = B // NWORKERS
CHUNKS_PER_ROW = L // 16
BBLK = 2048
GRID = B // BBLK

_SQRT_HALF = 0.7071067811865476

def _gelu(x):
    return 0.5 * x * (1.0 + lax.erf(x * _SQRT_HALF))


def _softplus(x):
    return jnp.maximum(x, 0.0) + jnp.log(1.0 + jnp.exp(-jnp.abs(x)))


def _silu(x):
    return x * (1.0 / (1.0 + jnp.exp(-x)))


def _tc_body(x_ref, c0w, c0b, c1w, c1b, c2w, c2b, c3w, c3b, c4w, c4b,
             pos_ref, lnb_ref, ipw_ref, mcw_ref, mcb_ref, xpw_ref,
             dtw_ref, dtb_ref, acat_ref, d_ref, opw_ref, emb_ref,
             res2_ref, m_ref, embb_ref, *wband):
    cws = (c0w, c1w, c2w, c3w, c4w)
    cbs = (c0b, c1b, c2b, c3b, c4b)
    f32 = jnp.float32
    dils = (1, 2, 4, 8, 16)

    @pl.when(pl.program_id(0) == 0)
    def _build_bands():
        ri = lax.broadcasted_iota(jnp.int32, (L, L), 0)
        ci = lax.broadcasted_iota(jnp.int32, (L, L), 1)
        diff = ri - ci
        for li in range(5):
            d = dils[li]
            w = jnp.zeros((L, L), f32)
            for k in range(5):
                w = jnp.where(diff == (k - 2) * d, cws[li][0, 0, k], w)
            wband[li][...] = w

    x = x_ref[...]
    h = x
    for li in range(5):
        acc = jnp.dot(h, wband[li][...], preferred_element_type=f32)
        acc = acc + cbs[li][0]
        h = _gelu(acc) if li < 4 else acc
    res2_ref[...] = h + x

    @pl.when(pl.program_id(0) == 0)
    def _compute_m():
        embb_ref[...] = jnp.broadcast_to(emb_ref[...], (K, 16))
        lnb = lnb_ref[0]
        xc0 = lnb * ipw_ref[0, 0]
        xc1 = lnb * ipw_ref[1, 0]
        zc0 = lnb * ipw_ref[2, 0]
        zc1 = lnb * ipw_ref[3, 0]
        tt = lax.broadcasted_iota(jnp.int32, (1, L), 1)

        def urow(dch, xc):
            w1 = mcw_ref[dch, 0, 1]
            w2 = mcw_ref[dch, 0, 2]
            w3 = mcw_ref[dch, 0, 3]
            sfull = mcw_ref[dch, 0, 0] + w1 + w2 + w3
            s = jnp.where(tt == 0, w3,
                          jnp.where(tt == 1, w2 + w3,
                                    jnp.where(tt == 2, w1 + w2 + w3, sfull)))
            xcr = s * xc + mcb_ref[dch]
            return _silu(xcr)

        u0 = urow(0, xc0)
        u1 = urow(1, xc1)
        xdbl = xpw_ref[:, 0:1] * u0 + xpw_ref[:, 1:2] * u1
        dtr = xdbl[0:1, :]
        Bm = xdbl[1:49, :]
        Cm = xdbl[49:97, :]
        dlt0 = _softplus(dtr * dtw_ref[0, 0] + dtb_ref[0])
        dlt1 = _softplus(dtr * dtw_ref[1, 0] + dtb_ref[1])
        d48_0 = jnp.broadcast_to(dlt0, (48, L))
        d48_1 = jnp.broadcast_to(dlt1, (48, L))
        dcat = jnp.concatenate([d48_0, d48_1, d48_0, d48_1], axis=0)
        u48_0 = jnp.broadcast_to(u0, (48, L))
        u48_1 = jnp.broadcast_to(u1, (48, L))
        ucat = jnp.concatenate([u48_0, u48_1, u48_0, u48_1], axis=0)
        a_coef = -jnp.exp(acat_ref[...])
        a = jnp.exp(dcat * a_coef)
        bt = jnp.concatenate([Bm, Bm, Bm, Bm], axis=0)
        b = dcat * bt * ucat
        s_ = 1
        while s_ < L:
            pad1 = jnp.ones((192, s_), f32)
            pad0 = jnp.zeros((192, s_), f32)
            a_sh = jnp.concatenate([pad1, a[:, :L - s_]], axis=1)
            b_sh = jnp.concatenate([pad0, b[:, :L - s_]], axis=1)
            b = a * b_sh + b
            a = a * a_sh
            s_ *= 2
        y0 = jnp.sum(b[0:48, :] * Cm, axis=0, keepdims=True)
        y1 = jnp.sum(b[48:96, :] * Cm, axis=0, keepdims=True)
        y2 = jnp.sum(b[96:144, :] * Cm, axis=0, keepdims=True)
        y3 = jnp.sum(b[144:192, :] * Cm, axis=0, keepdims=True)
        sz0 = _silu(jnp.full((1, 1), zc0, f32))
        sz1 = _silu(jnp.full((1, 1), zc1, f32))
        w0 = opw_ref[0, 0]
        w1 = opw_ref[0, 1]
        yf0 = (y0 + u0 * d_ref[0]) * sz0
        yf1 = (y1 + u1 * d_ref[1]) * sz1
        yb0 = (y2 + u0 * d_ref[0]) * sz0
        yb1 = (y3 + u1 * d_ref[1]) * sz1
        ri = lax.broadcasted_iota(jnp.int32, (L, L), 0)
        ci = lax.broadcasted_iota(jnp.int32, (L, L), 1)
        perm = jnp.where(ri + ci == L - 1, 1.0, 0.0)
        yb2 = jnp.concatenate([yb0, yb1], axis=0)
        ybr = jnp.dot(yb2, perm, preferred_element_type=f32)
        m2 = (yf0 + ybr[0:1, :]) * w0 + (yf1 + ybr[1:2, :]) * w1
        m_ref[...] = jnp.concatenate([m2, pos_ref[...]], axis=0)


def _sc_body(x_hbm, m_hbm, emb_hbm, quant_hbm, idx_hbm, loss_hbm,
             xb, mb, eb, qb, ib, lb):
    cid = lax.axis_index("c")
    sid = lax.axis_index("s")
    wid = sid * NCORES + cid
    row0 = wid * ROWS_PER_W
    pltpu.sync_copy(x_hbm.at[pl.ds(row0, ROWS_PER_W)], xb)
    pltpu.sync_copy(m_hbm, mb)
    pltpu.sync_copy(emb_hbm, eb)
    mrow = [mb[0, pl.ds(c * 16, 16)] for c in range(CHUNKS_PER_ROW)]
    prow = [mb[1, pl.ds(c * 16, 16)] for c in range(CHUNKS_PER_ROW)]

    def body(r, acc):
        for c in range(CHUNKS_PER_ROW):
            ds = pl.ds(c * 16, 16)
            xv = mrow[c] + (xb[r, ds] + prow[c])
            e0 = eb[0]
            bd = jnp.abs(xv - e0)
            bi = jnp.zeros((16,), jnp.int32)
            bq = e0
            for k in range(1, K):
                ek = eb[k]
                dk = jnp.abs(xv - ek)
                bet = dk < bd
                bd = jnp.where(bet, dk, bd)
                bi = jnp.where(bet, jnp.full((16,), k, jnp.int32), bi)
                bq = jnp.where(bet, ek, bq)
            qb[r, ds] = bq
            ib[r, ds] = bi
            df = bq - xv
            acc = acc + df * df
        return acc

    acc = lax.fori_loop(0, ROWS_PER_W, body, jnp.zeros((16,), jnp.float32))
    lb[...] = acc
    pltpu.sync_copy(qb, quant_hbm.at[pl.ds(row0, ROWS_PER_W)])
    pltpu.sync_copy(ib, idx_hbm.at[pl.ds(row0, ROWS_PER_W)])
    pltpu.sync_copy(lb, loss_hbm.at[wid])


def _smem_spec():
    return pl.BlockSpec(memory_space=pltpu.SMEM)


def _full_vmem(shape):
    return pl.BlockSpec(shape, lambda *_: tuple(0 for _ in shape))


_tc_call = pl.pallas_call(
    _tc_body,
    grid=(GRID,),
    in_specs=[
        pl.BlockSpec((BBLK, L), lambda i: (i, 0)),
        _smem_spec(), _smem_spec(),
        _smem_spec(), _smem_spec(),
        _smem_spec(), _smem_spec(),
        _smem_spec(), _smem_spec(),
        _smem_spec(), _smem_spec(),
        _full_vmem((1, L)),
        _smem_spec(),
        _smem_spec(),
        _smem_spec(),
        _smem_spec(),
        _full_vmem((97, 2)),
        _smem_spec(),
        _smem_spec(),
        _full_vmem((192, 1)),
        _smem_spec(),
        _smem_spec(),
        _full_vmem((K, 1)),
    ],
    out_specs=[
        pl.BlockSpec((BBLK, L), lambda i: (i, 0)),
        pl.BlockSpec((2, L), lambda i: (0, 0)),
        pl.BlockSpec((K, 16), lambda i: (0, 0)),
    ],
    out_shape=[
        jax.ShapeDtypeStruct((B, L), jnp.float32),
        jax.ShapeDtypeStruct((2, L), jnp.float32),
        jax.ShapeDtypeStruct((K, 16), jnp.float32),
    ],
    scratch_shapes=[pltpu.VMEM((L, L), jnp.float32) for _ in range(5)],
)


@functools.cache
def _get_sc_call():
    mesh = plsc.VectorSubcoreMesh(core_axis_name="c", subcore_axis_name="s",
                                  num_cores=NCORES, num_subcores=NSUB)
    return pl.kernel(
        _sc_body,
        out_type=[
            jax.ShapeDtypeStruct((B, L), jnp.float32),
            jax.ShapeDtypeStruct((B, L), jnp.int32),
            jax.ShapeDtypeStruct((NWORKERS, 16), jnp.float32),
        ],
        mesh=mesh,
        scratch_types=[
            pltpu.VMEM((ROWS_PER_W, L), jnp.float32),
            pltpu.VMEM((2, L), jnp.float32),
            pltpu.VMEM((K, 16), jnp.float32),
            pltpu.VMEM((ROWS_PER_W, L), jnp.float32),
            pltpu.VMEM((ROWS_PER_W, L), jnp.int32),
            pltpu.VMEM((16,), jnp.float32),
        ],
    )


def kernel(inputs, c0_w, c0_b, c1_w, c1_b, c2_w, c2_b, c3_w, c3_b, c4_w,
           c4_b, pos_emb, ln_w, ln_b, in_proj_w, conv1d_w, conv1d_b,
           x_proj_w, dt_proj_w, dt_proj_b, A_log, A_b_log, D, out_proj_w,
           emb):
    del ln_w
    acat = jnp.concatenate(
        [A_log.reshape(-1), A_b_log.reshape(-1)]).reshape(2 * 2 * 48, 1)
    res2, m_plus, embb = _tc_call(
        inputs,
        c0_w, c0_b, c1_w, c1_b, c2_w, c2_b, c3_w, c3_b, c4_w, c4_b,
        pos_emb.reshape(1, L), ln_b, in_proj_w, conv1d_w, conv1d_b,
        x_proj_w, dt_proj_w, dt_proj_b, acat, D, out_proj_w, emb)
    quant, idx, loss_part = _get_sc_call()(res2, m_plus, embb)
    c_loss = 0.5 * (jnp.sum(loss_part) / (B * L))
    return c_loss[None], quant, idx
